# baseline (speedup 1.0000x reference)
import functools

import jax
import jax.numpy as jnp
from jax import lax
from jax.experimental import pallas as pl
from jax.experimental.pallas import tpu as pltpu
from jax.experimental.pallas import tpu_sc as plsc

B = 16384
F_CAT = 26
F_CONT = 13
V = 100000
ROW = V + 1
D = 32
EPS = 1e-5

_IDX_BLK = 2048


def _idx_body(x_ref, idx_ref):
    x = x_ref[...]
    cat = x[:, :F_CAT].astype(jnp.int32)
    offs = lax.broadcasted_iota(jnp.int32, (1, F_CAT), 1) * ROW
    idx_ref[...] = cat + offs


def _make_idx_call():
    return pl.pallas_call(
        _idx_body,
        grid=(B // _IDX_BLK,),
        in_specs=[pl.BlockSpec((_IDX_BLK, F_CAT + F_CONT), lambda i: (i, 0))],
        out_specs=pl.BlockSpec((_IDX_BLK, F_CAT), lambda i: (i, 0)),
        out_shape=jax.ShapeDtypeStruct((B, F_CAT), jnp.int32),
    )


_NC = 2
_NS = 16
_NW = _NC * _NS
_RPW = B * F_CAT // _NW
_CH = 128
_NCH = _RPW // _CH


def _sc_gather_body(tbl, idx, out, idx_v, buf, sem):
    wid = lax.axis_index("s") * _NC + lax.axis_index("c")
    pltpu.sync_copy(idx.at[pl.ds(wid * _NCH, _NCH)], idx_v)

    def body(j, carry):
        pltpu.async_copy(tbl.at[idx_v.at[j]], buf, sem).wait()
        pltpu.sync_copy(buf, out.at[pl.ds(wid * _RPW + j * _CH, _CH)])
        return carry

    lax.fori_loop(0, _NCH, body, 0)


def _make_sc_gather():
    mesh = plsc.VectorSubcoreMesh(core_axis_name="c", subcore_axis_name="s")
    return functools.partial(
        pl.kernel,
        mesh=mesh,
        compiler_params=pltpu.CompilerParams(use_tc_tiling_on_sc=False),
        out_type=jax.ShapeDtypeStruct((B * F_CAT, D), jnp.float32),
        scratch_types=[
            pltpu.VMEM((_NCH, _CH), jnp.int32),
            pltpu.VMEM((_CH, D), jnp.float32),
            pltpu.SemaphoreType.DMA,
        ],
    )(_sc_gather_body)


_ASM_BLK = 1024


def _asm_body(emb_ref, x_ref, gamma_ref, beta_ref, out_ref):
    xc = x_ref[...][:, F_CAT:]
    mu = jnp.mean(xc, axis=-1, keepdims=True)
    var = jnp.mean((xc - mu) * (xc - mu), axis=-1, keepdims=True)
    xcn = (xc - mu) * lax.rsqrt(var + EPS)
    xcn = xcn * gamma_ref[...] + beta_ref[...]
    out_ref[...] = jnp.concatenate([emb_ref[...], xcn], axis=1)


def _make_asm_call():
    return pl.pallas_call(
        _asm_body,
        grid=(B // _ASM_BLK,),
        in_specs=[
            pl.BlockSpec((_ASM_BLK, F_CAT * D), lambda i: (i, 0)),
            pl.BlockSpec((_ASM_BLK, F_CAT + F_CONT), lambda i: (i, 0)),
            pl.BlockSpec((1, F_CONT), lambda i: (0, 0)),
            pl.BlockSpec((1, F_CONT), lambda i: (0, 0)),
        ],
        out_specs=pl.BlockSpec((_ASM_BLK, F_CAT * D + F_CONT), lambda i: (i, 0)),
        out_shape=jax.ShapeDtypeStruct((B, F_CAT * D + F_CONT), jnp.float32),
    )


@jax.jit
def kernel(X, tables, gamma, beta):
    idx = _make_idx_call()(X)
    idx2 = idx.reshape(B * F_CAT // _CH, _CH)
    tbl = tables.reshape(F_CAT * ROW, D)
    emb = _make_sc_gather()(tbl, idx2)
    embr = emb.reshape(B, F_CAT * D)
    return _make_asm_call()(
        embr, X, gamma.reshape(1, F_CONT), beta.reshape(1, F_CONT)
    )


# speedup vs baseline: 21.3825x; 21.3825x over previous
import functools

import jax
import jax.numpy as jnp
from jax import lax
from jax.experimental import pallas as pl
from jax.experimental.pallas import tpu as pltpu
from jax.experimental.pallas import tpu_sc as plsc

B = 16384
F_CAT = 26
F_CONT = 13
V = 100000
ROW = V + 1
D = 32
EPS = 1e-5

_IDX_BLK = 2048


def _idx_body(x_ref, idx_ref):
    x = x_ref[...]
    idx_ref[...] = x[:, :F_CAT].astype(jnp.int32).T


def _make_idx_call():
    return pl.pallas_call(
        _idx_body,
        grid=(B // _IDX_BLK,),
        in_specs=[pl.BlockSpec((_IDX_BLK, F_CAT + F_CONT), lambda i: (i, 0))],
        out_specs=pl.BlockSpec((F_CAT, _IDX_BLK), lambda i: (0, i)),
        out_shape=jax.ShapeDtypeStruct((F_CAT, B), jnp.int32),
    )


_NC = 2
_NS = 16
_NW = _NC * _NS
_PAIRS = F_CAT * D
_PPW = _PAIRS // _NW
_OCH = 4096
_LANES = 16


def _sc_gather_body(tbl, idx, out, vec_v, idx_v, out_v):
    wid = lax.axis_index("s") * _NC + lax.axis_index("c")

    def pair_body(t, carry):
        p = wid * _PPW + t
        f = p // D
        d = p - f * D
        pltpu.sync_copy(tbl.at[f, d], vec_v)
        pltpu.sync_copy(idx.at[f], idx_v)

        def chunk_body(c, carry2):
            def gather_body(k, carry3):
                iv = idx_v[pl.ds(c * _OCH + k * _LANES, _LANES)]
                out_v[pl.ds(k * _LANES, _LANES)] = plsc.load_gather(
                    vec_v, [iv]
                )
                return carry3

            lax.fori_loop(0, _OCH // _LANES, gather_body, 0, unroll=4)
            pltpu.sync_copy(out_v, out.at[p, pl.ds(c * _OCH, _OCH)])
            return carry2

        lax.fori_loop(0, B // _OCH, chunk_body, 0)
        return carry

    lax.fori_loop(0, _PPW, pair_body, 0)


def _make_sc_gather():
    mesh = plsc.VectorSubcoreMesh(core_axis_name="c", subcore_axis_name="s")
    return functools.partial(
        pl.kernel,
        mesh=mesh,
        compiler_params=pltpu.CompilerParams(needs_layout_passes=False),
        out_type=jax.ShapeDtypeStruct((_PAIRS, B), jnp.float32),
        scratch_types=[
            pltpu.VMEM((ROW,), jnp.float32),
            pltpu.VMEM((B,), jnp.int32),
            pltpu.VMEM((_OCH,), jnp.float32),
        ],
    )(_sc_gather_body)


_ASM_BLK = 256


def _asm_body(emb_ref, x_ref, gamma_ref, beta_ref, out_ref):
    et = emb_ref[...].T
    xc = x_ref[...][:, F_CAT:]
    mu = jnp.mean(xc, axis=-1, keepdims=True)
    var = jnp.mean((xc - mu) * (xc - mu), axis=-1, keepdims=True)
    xcn = (xc - mu) * lax.rsqrt(var + EPS)
    xcn = xcn * gamma_ref[...] + beta_ref[...]
    out_ref[...] = jnp.concatenate([et, xcn], axis=1)


def _make_asm_call():
    return pl.pallas_call(
        _asm_body,
        grid=(B // _ASM_BLK,),
        in_specs=[
            pl.BlockSpec((_PAIRS, _ASM_BLK), lambda i: (0, i)),
            pl.BlockSpec((_ASM_BLK, F_CAT + F_CONT), lambda i: (i, 0)),
            pl.BlockSpec((1, F_CONT), lambda i: (0, 0)),
            pl.BlockSpec((1, F_CONT), lambda i: (0, 0)),
        ],
        out_specs=pl.BlockSpec((_ASM_BLK, _PAIRS + F_CONT), lambda i: (i, 0)),
        out_shape=jax.ShapeDtypeStruct((B, _PAIRS + F_CONT), jnp.float32),
    )


@jax.jit
def kernel(X, tables, gamma, beta):
    tbl_t = jnp.swapaxes(tables, 1, 2)
    idx_t = _make_idx_call()(X)
    emb_t = _make_sc_gather()(tbl_t, idx_t)
    return _make_asm_call()(
        emb_t, X, gamma.reshape(1, F_CONT), beta.reshape(1, F_CONT)
    )


# speedup vs baseline: 44.5814x; 2.0849x over previous
import functools

import jax
import jax.numpy as jnp
from jax import lax
from jax.experimental import pallas as pl
from jax.experimental.pallas import tpu as pltpu
from jax.experimental.pallas import tpu_sc as plsc

B = 16384
F_CAT = 26
F_CONT = 13
V = 100000
ROW = V + 1
D = 32
EPS = 1e-5

_NC = 2
_NS = 16
_NW = _NC * _NS
_PAIRS = F_CAT * D
_PPW = _PAIRS // _NW
_OCH = 4096
_LANES = 16


def _sc_gather_body(tbl, idx, out, vec_v, idx_v, outa_v, outb_v, osem):
    wid = lax.axis_index("s") * _NC + lax.axis_index("c")
    obufs = (outa_v, outb_v)

    def pair_body(t, carry):
        p = wid * _PPW + t
        f = p // D
        d = p - f * D
        pltpu.sync_copy(tbl.at[f, d], vec_v)

        @pl.when(jnp.logical_or(p % D == 0, t == 0))
        def _():
            pltpu.sync_copy(idx.at[f], idx_v)

        for c in range(B // _OCH):
            ob = obufs[c % 2]
            if c >= 2:
                pltpu.make_async_copy(
                    out.at[0, pl.ds(0, _OCH)], ob, osem
                ).wait()
            else:

                @pl.when(t > 0)
                def _():
                    pltpu.make_async_copy(
                        out.at[0, pl.ds(0, _OCH)], ob, osem
                    ).wait()

            @plsc.parallel_loop(0, _OCH // _LANES, unroll=16)
            def gather_body(k):
                iv = idx_v[pl.ds(c * _OCH + k * _LANES, _LANES)]
                ob[pl.ds(k * _LANES, _LANES)] = plsc.load_gather(
                    vec_v, [iv]
                )

            pltpu.async_copy(ob, out.at[p, pl.ds(c * _OCH, _OCH)], osem)
        return carry

    lax.fori_loop(0, _PPW, pair_body, 0)
    for _ in range(2):
        pltpu.make_async_copy(out.at[0, pl.ds(0, _OCH)], outa_v, osem).wait()


def _make_sc_gather():
    mesh = plsc.VectorSubcoreMesh(core_axis_name="c", subcore_axis_name="s")
    return functools.partial(
        pl.kernel,
        mesh=mesh,
        compiler_params=pltpu.CompilerParams(needs_layout_passes=False),
        out_type=jax.ShapeDtypeStruct((_PAIRS, B), jnp.float32),
        scratch_types=[
            pltpu.VMEM((ROW,), jnp.float32),
            pltpu.VMEM((B,), jnp.int32),
            pltpu.VMEM((_OCH,), jnp.float32),
            pltpu.VMEM((_OCH,), jnp.float32),
            pltpu.SemaphoreType.DMA,
        ],
    )(_sc_gather_body)


_ASM_BLK = 2048


def _asm_body(emb_ref, x_ref, gamma_ref, beta_ref, out_ref):
    et = emb_ref[...].T
    xc = x_ref[...][:, F_CAT:]
    mu = jnp.mean(xc, axis=-1, keepdims=True)
    var = jnp.mean((xc - mu) * (xc - mu), axis=-1, keepdims=True)
    xcn = (xc - mu) * lax.rsqrt(var + EPS)
    xcn = xcn * gamma_ref[...] + beta_ref[...]
    out_ref[...] = jnp.concatenate([et, xcn], axis=1)


def _make_asm_call():
    return pl.pallas_call(
        _asm_body,
        grid=(B // _ASM_BLK,),
        in_specs=[
            pl.BlockSpec((_PAIRS, _ASM_BLK), lambda i: (0, i)),
            pl.BlockSpec((_ASM_BLK, F_CAT + F_CONT), lambda i: (i, 0)),
            pl.BlockSpec((1, F_CONT), lambda i: (0, 0)),
            pl.BlockSpec((1, F_CONT), lambda i: (0, 0)),
        ],
        out_specs=pl.BlockSpec((_ASM_BLK, _PAIRS + F_CONT), lambda i: (i, 0)),
        out_shape=jax.ShapeDtypeStruct((B, _PAIRS + F_CONT), jnp.float32),
    )


@jax.jit
def kernel(X, tables, gamma, beta):
    tbl_t = jnp.swapaxes(tables, 1, 2)
    idx_t = X[:, :F_CAT].astype(jnp.int32).T.copy()
    emb_t = _make_sc_gather()(tbl_t, idx_t)
    return _make_asm_call()(
        emb_t, X, gamma.reshape(1, F_CONT), beta.reshape(1, F_CONT)
    )


# speedup vs baseline: 62.3385x; 1.3983x over previous
import functools

import jax
import jax.numpy as jnp
from jax import lax
from jax.experimental import pallas as pl
from jax.experimental.pallas import tpu as pltpu
from jax.experimental.pallas import tpu_sc as plsc

B = 16384
F_CAT = 26
F_CONT = 13
V = 100000
ROW = V + 1
D = 32
EPS = 1e-5

_NC = 2
_NS = 16
_NW = _NC * _NS
_PAIRS = F_CAT * D
_PPW = _PAIRS // _NW
_OCH = 4096
_LANES = 16


def _sc_gather_body(tbl, idx, out, vec_v, idx_v, outa_v, outb_v, osem):
    wid = lax.axis_index("s") * _NC + lax.axis_index("c")
    obufs = (outa_v, outb_v)

    def pair_body(t, carry):
        p = wid * _PPW + t
        f = p // D
        d = p - f * D
        pltpu.sync_copy(tbl.at[f, d], vec_v)

        @pl.when(jnp.logical_or(p % D == 0, t == 0))
        def _():
            pltpu.sync_copy(idx.at[f], idx_v)

        for c in range(B // _OCH):
            ob = obufs[c % 2]
            if c >= 2:
                pltpu.make_async_copy(
                    out.at[0, pl.ds(0, _OCH)], ob, osem
                ).wait()
            else:

                @pl.when(t > 0)
                def _():
                    pltpu.make_async_copy(
                        out.at[0, pl.ds(0, _OCH)], ob, osem
                    ).wait()

            @plsc.parallel_loop(0, _OCH // _LANES, unroll=16)
            def gather_body(k):
                iv = idx_v[pl.ds(c * _OCH + k * _LANES, _LANES)]
                ob[pl.ds(k * _LANES, _LANES)] = plsc.load_gather(
                    vec_v, [iv]
                )

            pltpu.async_copy(ob, out.at[p, pl.ds(c * _OCH, _OCH)], osem)
        return carry

    lax.fori_loop(0, _PPW, pair_body, 0)
    for _ in range(2):
        pltpu.make_async_copy(out.at[0, pl.ds(0, _OCH)], outa_v, osem).wait()


def _make_sc_gather():
    mesh = plsc.VectorSubcoreMesh(core_axis_name="c", subcore_axis_name="s")
    return functools.partial(
        pl.kernel,
        mesh=mesh,
        compiler_params=pltpu.CompilerParams(needs_layout_passes=False),
        out_type=jax.ShapeDtypeStruct((_PAIRS + F_CONT, B), jnp.float32),
        scratch_types=[
            pltpu.VMEM((ROW,), jnp.float32),
            pltpu.VMEM((B,), jnp.int32),
            pltpu.VMEM((_OCH,), jnp.float32),
            pltpu.VMEM((_OCH,), jnp.float32),
            pltpu.SemaphoreType.DMA,
        ],
    )(_sc_gather_body)


_LN_BLK = 2048


def _ln_body(o_ref, x_ref, gamma_ref, beta_ref, out_ref):
    xc = x_ref[...][:, F_CAT:]
    mu = jnp.mean(xc, axis=-1, keepdims=True)
    var = jnp.mean((xc - mu) * (xc - mu), axis=-1, keepdims=True)
    xcn = (xc - mu) * lax.rsqrt(var + EPS)
    xcn = xcn * gamma_ref[...] + beta_ref[...]
    pad = jnp.zeros((3, _LN_BLK), jnp.float32)
    out_ref[...] = jnp.concatenate([xcn.T, pad], axis=0)


def _make_ln_call():
    return pl.pallas_call(
        _ln_body,
        grid=(B // _LN_BLK,),
        in_specs=[
            pl.BlockSpec((16, _LN_BLK), lambda i: (_PAIRS // 16, i)),
            pl.BlockSpec((_LN_BLK, F_CAT + F_CONT), lambda i: (i, 0)),
            pl.BlockSpec((1, F_CONT), lambda i: (0, 0)),
            pl.BlockSpec((1, F_CONT), lambda i: (0, 0)),
        ],
        out_specs=pl.BlockSpec((16, _LN_BLK), lambda i: (_PAIRS // 16, i)),
        out_shape=jax.ShapeDtypeStruct((_PAIRS + F_CONT, B), jnp.float32),
        input_output_aliases={0: 0},
    )


@jax.jit
def kernel(X, tables, gamma, beta):
    tbl_t = jnp.swapaxes(tables, 1, 2)
    idx_t = X[:, :F_CAT].astype(jnp.int32).T.copy()
    out_t = _make_sc_gather()(tbl_t, idx_t)
    out_t = _make_ln_call()(
        out_t, X, gamma.reshape(1, F_CONT), beta.reshape(1, F_CONT)
    )
    return out_t.T
